# Initial kernel scaffold; baseline (speedup 1.0000x reference)
import jax
import jax.numpy as jnp
from jax.experimental import pallas as pl


def kernel(x, c0, c1, c2):
    raise NotImplementedError("write your pallas kernel here")


# speedup vs baseline: 1.5837x; 1.5837x over previous
import jax
import jax.numpy as jnp
from jax.experimental import pallas as pl

_N, _D, _K = 16384, 256, 1024
_BM = 1024


def _rq_body(x_ref, c0_ref, c1_ref, c2_ref, p0_ref, p1_ref, p2_ref, out_ref):
    r = x_ref[...]
    recon = jnp.zeros_like(r)
    r2 = jnp.sum(r * r, axis=1, keepdims=True)
    for c_ref, p_ref in ((c0_ref, p0_ref), (c1_ref, p1_ref), (c2_ref, p2_ref)):
        c = c_ref[...]
        c2 = jnp.sum(c * c, axis=1)[None, :]
        rc = jax.lax.dot_general(
            r.astype(jnp.bfloat16), c.astype(jnp.bfloat16),
            (((1,), (1,)), ((), ())),
            preferred_element_type=jnp.float32)
        dist2 = (r2 + c2) - 2.0 * rc
        idx = jnp.argmin(dist2, axis=1)
        oh = (jax.lax.broadcasted_iota(jnp.int32, dist2.shape, 1)
              == idx[:, None]).astype(jnp.int8)
        planes = []
        for b in range(4):
            pb = jax.lax.dot_general(
                oh, p_ref[b], (((1,), (0,)), ((), ())),
                preferred_element_type=jnp.int32)
            planes.append(pb)
        word = ((planes[0] & 255)
                | ((planes[1] & 255) << 8)
                | ((planes[2] & 255) << 16)
                | ((planes[3] & 255) << 24))
        q = jax.lax.bitcast_convert_type(word, jnp.float32)
        recon = recon + q
        r = r - q
        r2 = jnp.sum(r * r, axis=1, keepdims=True)
    out_ref[...] = recon


def _byte_planes(c):
    b = jax.lax.bitcast_convert_type(c, jnp.int8)
    return jnp.transpose(b, (2, 0, 1))


def _call(x, c0, c1, c2, p0, p1, p2, *, interpret=False):
    cspec = pl.BlockSpec((_K, _D), lambda i: (0, 0))
    pspec = pl.BlockSpec((4, _K, _D), lambda i: (0, 0, 0))
    return pl.pallas_call(
        _rq_body,
        grid=(_N // _BM,),
        in_specs=[pl.BlockSpec((_BM, _D), lambda i: (i, 0)),
                  cspec, cspec, cspec, pspec, pspec, pspec],
        out_specs=pl.BlockSpec((_BM, _D), lambda i: (i, 0)),
        out_shape=jax.ShapeDtypeStruct((_N, _D), jnp.float32),
        interpret=interpret,
    )(x, c0, c1, c2, p0, p1, p2)


@jax.jit
def kernel(x, c0, c1, c2):
    return _call(x, c0, c1, c2,
                 _byte_planes(c0), _byte_planes(c1), _byte_planes(c2))


# speedup vs baseline: 1.6764x; 1.0585x over previous
import jax
import jax.numpy as jnp
from jax.experimental import pallas as pl

_N, _D, _K = 16384, 256, 1024
_BM = 1024
_NS = 2


def _layer(r, r2, c_ref, p_ref):
    c = c_ref[...]
    c2 = jnp.sum(c * c, axis=1)[None, :]
    rc = jax.lax.dot_general(
        r.astype(jnp.bfloat16), c.astype(jnp.bfloat16),
        (((1,), (1,)), ((), ())),
        preferred_element_type=jnp.float32)
    dist2 = (r2 + c2) - 2.0 * rc
    m = jnp.min(dist2, axis=1, keepdims=True)
    iota_f = jax.lax.broadcasted_iota(
        jnp.int32, dist2.shape, 1).astype(jnp.float32)
    idxf = jnp.min(jnp.where(dist2 == m, iota_f, jnp.float32(_K)),
                   axis=1, keepdims=True)
    oh = (iota_f == idxf).astype(jnp.int8)
    planes = []
    for b in range(4):
        pb = jax.lax.dot_general(
            oh, p_ref[b], (((1,), (0,)), ((), ())),
            preferred_element_type=jnp.int32)
        planes.append(pb)
    word = ((planes[0] & 255)
            | ((planes[1] & 255) << 8)
            | ((planes[2] & 255) << 16)
            | ((planes[3] & 255) << 24))
    return jax.lax.bitcast_convert_type(word, jnp.float32)


def _rq_body(x_ref, c0_ref, c1_ref, c2_ref, p0_ref, p1_ref, p2_ref, out_ref):
    sm = _BM // _NS
    rs = [x_ref[pl.ds(s * sm, sm), :] for s in range(_NS)]
    recons = [jnp.zeros_like(r) for r in rs]
    r2s = [jnp.sum(r * r, axis=1, keepdims=True) for r in rs]
    for c_ref, p_ref in ((c0_ref, p0_ref), (c1_ref, p1_ref), (c2_ref, p2_ref)):
        for s in range(_NS):
            q = _layer(rs[s], r2s[s], c_ref, p_ref)
            recons[s] = recons[s] + q
            rs[s] = rs[s] - q
            r2s[s] = jnp.sum(rs[s] * rs[s], axis=1, keepdims=True)
    for s in range(_NS):
        out_ref[pl.ds(s * sm, sm), :] = recons[s]


def _byte_planes(c):
    b = jax.lax.bitcast_convert_type(c, jnp.int8)
    return jnp.transpose(b, (2, 0, 1))


def _call(x, c0, c1, c2, p0, p1, p2, *, interpret=False):
    cspec = pl.BlockSpec((_K, _D), lambda i: (0, 0))
    pspec = pl.BlockSpec((4, _K, _D), lambda i: (0, 0, 0))
    return pl.pallas_call(
        _rq_body,
        grid=(_N // _BM,),
        in_specs=[pl.BlockSpec((_BM, _D), lambda i: (i, 0)),
                  cspec, cspec, cspec, pspec, pspec, pspec],
        out_specs=pl.BlockSpec((_BM, _D), lambda i: (i, 0)),
        out_shape=jax.ShapeDtypeStruct((_N, _D), jnp.float32),
        interpret=interpret,
    )(x, c0, c1, c2, p0, p1, p2)


@jax.jit
def kernel(x, c0, c1, c2):
    return _call(x, c0, c1, c2,
                 _byte_planes(c0), _byte_planes(c1), _byte_planes(c2))


# speedup vs baseline: 2.1232x; 1.2665x over previous
import jax
import jax.numpy as jnp
from jax.experimental import pallas as pl

_N, _D, _K = 16384, 256, 1024
_BM = 1024
_NS = 2


def _layer(r, r2, c_ref, s_ref):
    c = c_ref[...]
    c2 = jnp.sum(c * c, axis=1)[None, :]
    rc = jax.lax.dot_general(
        r.astype(jnp.bfloat16), c.astype(jnp.bfloat16),
        (((1,), (1,)), ((), ())),
        preferred_element_type=jnp.float32)
    dist2 = (r2 + c2) - 2.0 * rc
    m = jnp.min(dist2, axis=1, keepdims=True)
    iota_f = jax.lax.broadcasted_iota(
        jnp.int32, dist2.shape, 1).astype(jnp.float32)
    idxf = jnp.min(jnp.where(dist2 == m, iota_f, jnp.float32(_K)),
                   axis=1, keepdims=True)
    oh = (iota_f == idxf).astype(jnp.bfloat16)
    parts = [
        jax.lax.dot_general(
            oh, s_ref[t], (((1,), (0,)), ((), ())),
            preferred_element_type=jnp.float32)
        for t in range(3)
    ]
    return (parts[0] + parts[1]) + parts[2]


def _rq_body(x_ref, c0_ref, c1_ref, c2_ref, s0_ref, s1_ref, s2_ref, out_ref):
    sm = _BM // _NS
    rs = [x_ref[pl.ds(s * sm, sm), :] for s in range(_NS)]
    recons = [jnp.zeros_like(r) for r in rs]
    r2s = [jnp.sum(r * r, axis=1, keepdims=True) for r in rs]
    for c_ref, s_ref in ((c0_ref, s0_ref), (c1_ref, s1_ref), (c2_ref, s2_ref)):
        for s in range(_NS):
            q = _layer(rs[s], r2s[s], c_ref, s_ref)
            recons[s] = recons[s] + q
            rs[s] = rs[s] - q
            r2s[s] = jnp.sum(rs[s] * rs[s], axis=1, keepdims=True)
    for s in range(_NS):
        out_ref[pl.ds(s * sm, sm), :] = recons[s]


def _bf16_split3(c):
    t0 = c.astype(jnp.bfloat16)
    d1 = c - t0.astype(jnp.float32)
    t1 = d1.astype(jnp.bfloat16)
    t2 = (d1 - t1.astype(jnp.float32)).astype(jnp.bfloat16)
    return jnp.stack([t0, t1, t2])


def _call(x, c0, c1, c2, s0, s1, s2, *, interpret=False):
    cspec = pl.BlockSpec((_K, _D), lambda i: (0, 0))
    sspec = pl.BlockSpec((3, _K, _D), lambda i: (0, 0, 0))
    return pl.pallas_call(
        _rq_body,
        grid=(_N // _BM,),
        in_specs=[pl.BlockSpec((_BM, _D), lambda i: (i, 0)),
                  cspec, cspec, cspec, sspec, sspec, sspec],
        out_specs=pl.BlockSpec((_BM, _D), lambda i: (i, 0)),
        out_shape=jax.ShapeDtypeStruct((_N, _D), jnp.float32),
        interpret=interpret,
    )(x, c0, c1, c2, s0, s1, s2)


@jax.jit
def kernel(x, c0, c1, c2):
    return _call(x, c0, c1, c2,
                 _bf16_split3(c0), _bf16_split3(c1), _bf16_split3(c2))
